# baseline (speedup 1.0000x reference)
import jax
import jax.numpy as jnp
from jax.experimental import pallas as pl
from jax.experimental.pallas import tpu as pltpu

B = 16384
NUM_CLASSES = 100
WORDS_PER_CLASS = 3
TOPK = 5
VOCAB = 100000
GLOVE_D = 300
FEAT = 1236
D_OUT = 1024

BLK = 1024


VBLK = 4000


def _gather_body(cwi_ref, w2v_ref, table_ref):
    i = pl.program_id(0)

    @pl.when(i == 0)
    def _():
        table_ref[...] = jnp.zeros_like(table_ref)

    base = i * VBLK
    cwi = cwi_ref[...]
    mask = (cwi != 0).astype(jnp.float32) * (1.0 / WORDS_PER_CLASS)
    iota = jax.lax.broadcasted_iota(jnp.int32, (NUM_CLASSES, VBLK), 1)
    m = jnp.zeros((NUM_CLASSES, VBLK), jnp.float32)
    for k in range(WORDS_PER_CLASS):
        m += (cwi[:, k:k + 1] - base == iota).astype(jnp.float32) * mask[:, k:k + 1]
    table_ref[...] += jnp.dot(m, w2v_ref[...],
                              preferred_element_type=jnp.float32)


def _main_body(ce_ref, lf_ref, table_ref, w1_ref, w2_ref, b_ref,
               out_ref, cls_out_ref):
    i = pl.program_id(0)

    @pl.when(i == 0)
    def _():
        cls_out_ref[...] = (
            jnp.dot(table_ref[...], w2_ref[...],
                    preferred_element_type=jnp.float32)
            + b_ref[...]
        )

    ce = ce_ref[...]
    iota = jax.lax.broadcasted_iota(jnp.int32, (BLK, NUM_CLASSES), 1)
    counts = jnp.zeros((BLK, NUM_CLASSES), jnp.float32)
    for k in range(TOPK):
        counts += (ce[:, k][:, None] == iota).astype(jnp.float32)
    counts = counts * (1.0 / TOPK)
    out_ref[...] = (
        jnp.dot(lf_ref[...], w1_ref[...], preferred_element_type=jnp.float32)
        + jnp.dot(counts, cls_out_ref[...], preferred_element_type=jnp.float32)
    )


def kernel(layers_feature, classes_embed, class_word_indices, word2vec, W, b):
    table = pl.pallas_call(
        _gather_body,
        grid=(VOCAB // VBLK,),
        in_specs=[
            pl.BlockSpec((NUM_CLASSES, WORDS_PER_CLASS), lambda i: (0, 0)),
            pl.BlockSpec((VBLK, GLOVE_D), lambda i: (i, 0)),
        ],
        out_specs=pl.BlockSpec((NUM_CLASSES, GLOVE_D), lambda i: (0, 0)),
        out_shape=jax.ShapeDtypeStruct((NUM_CLASSES, GLOVE_D), jnp.float32),
    )(class_word_indices, word2vec)

    W1 = W[:FEAT]
    W2 = W[FEAT:]
    b2 = b.reshape(1, D_OUT)

    out = pl.pallas_call(
        _main_body,
        grid=(B // BLK,),
        in_specs=[
            pl.BlockSpec((BLK, TOPK), lambda i: (i, 0)),
            pl.BlockSpec((BLK, FEAT), lambda i: (i, 0)),
            pl.BlockSpec((NUM_CLASSES, GLOVE_D), lambda i: (0, 0)),
            pl.BlockSpec((FEAT, D_OUT), lambda i: (0, 0)),
            pl.BlockSpec((GLOVE_D, D_OUT), lambda i: (0, 0)),
            pl.BlockSpec((1, D_OUT), lambda i: (0, 0)),
        ],
        out_specs=pl.BlockSpec((BLK, D_OUT), lambda i: (i, 0)),
        out_shape=jax.ShapeDtypeStruct((B, D_OUT), jnp.float32),
        scratch_shapes=[pltpu.VMEM((NUM_CLASSES, D_OUT), jnp.float32)],
    )(classes_embed, layers_feature, table, W1, W2, b2)
    return out


# speedup vs baseline: 1.1471x; 1.1471x over previous
import jax
import jax.numpy as jnp
from jax.experimental import pallas as pl
from jax.experimental.pallas import tpu as pltpu

B = 16384
NUM_CLASSES = 100
WORDS_PER_CLASS = 3
TOPK = 5
VOCAB = 100000
GLOVE_D = 300
FEAT = 1236
D_OUT = 1024

BLK = 1024


def _gather_body(idx_ref, cwi_ref, w2v_ref, table_ref, buf_ref, sem_ref):
    for c in range(NUM_CLASSES):
        for k in range(WORDS_PER_CLASS):
            w = idx_ref[c * WORDS_PER_CLASS + k]
            pltpu.make_async_copy(
                w2v_ref.at[pl.ds(w, 1), :], buf_ref.at[k, pl.ds(c, 1), :],
                sem_ref.at[k, c]).start()
    for c in range(NUM_CLASSES):
        for k in range(WORDS_PER_CLASS):
            pltpu.make_async_copy(
                w2v_ref.at[pl.ds(0, 1), :], buf_ref.at[k, pl.ds(c, 1), :],
                sem_ref.at[k, c]).wait()
    m = (cwi_ref[...] != 0).astype(jnp.float32) * (1.0 / WORDS_PER_CLASS)
    acc = buf_ref[0] * m[:, 0:1]
    acc += buf_ref[1] * m[:, 1:2]
    acc += buf_ref[2] * m[:, 2:3]
    table_ref[...] = acc


def _main_body(ce_ref, lf_ref, table_ref, w1_ref, w2_ref, b_ref,
               out_ref, cls_out_ref):
    i = pl.program_id(0)

    @pl.when(i == 0)
    def _():
        cls_out_ref[...] = (
            jnp.dot(table_ref[...], w2_ref[...],
                    preferred_element_type=jnp.float32)
            + b_ref[...]
        )

    ce = ce_ref[...]
    iota = jax.lax.broadcasted_iota(jnp.int32, (BLK, NUM_CLASSES), 1)
    counts = jnp.zeros((BLK, NUM_CLASSES), jnp.float32)
    for k in range(TOPK):
        counts += (ce[:, k][:, None] == iota).astype(jnp.float32)
    counts = counts * (1.0 / TOPK)
    out_ref[...] = (
        jnp.dot(lf_ref[...], w1_ref[...], preferred_element_type=jnp.float32)
        + jnp.dot(counts, cls_out_ref[...], preferred_element_type=jnp.float32)
    )


def kernel(layers_feature, classes_embed, class_word_indices, word2vec, W, b):
    idx_flat = class_word_indices.reshape(-1)

    table = pl.pallas_call(
        _gather_body,
        in_specs=[
            pl.BlockSpec(memory_space=pltpu.MemorySpace.SMEM),
            pl.BlockSpec(memory_space=pltpu.MemorySpace.VMEM),
            pl.BlockSpec(memory_space=pltpu.MemorySpace.HBM),
        ],
        out_specs=pl.BlockSpec(memory_space=pltpu.MemorySpace.VMEM),
        out_shape=jax.ShapeDtypeStruct((NUM_CLASSES, GLOVE_D), jnp.float32),
        scratch_shapes=[
            pltpu.VMEM((WORDS_PER_CLASS, NUM_CLASSES, GLOVE_D), jnp.float32),
            pltpu.SemaphoreType.DMA((WORDS_PER_CLASS, NUM_CLASSES)),
        ],
    )(idx_flat, class_word_indices, word2vec)

    W1 = W[:FEAT]
    W2 = W[FEAT:]
    b2 = b.reshape(1, D_OUT)

    out = pl.pallas_call(
        _main_body,
        grid=(B // BLK,),
        in_specs=[
            pl.BlockSpec((BLK, TOPK), lambda i: (i, 0)),
            pl.BlockSpec((BLK, FEAT), lambda i: (i, 0)),
            pl.BlockSpec((NUM_CLASSES, GLOVE_D), lambda i: (0, 0)),
            pl.BlockSpec((FEAT, D_OUT), lambda i: (0, 0)),
            pl.BlockSpec((GLOVE_D, D_OUT), lambda i: (0, 0)),
            pl.BlockSpec((1, D_OUT), lambda i: (0, 0)),
        ],
        out_specs=pl.BlockSpec((BLK, D_OUT), lambda i: (i, 0)),
        out_shape=jax.ShapeDtypeStruct((B, D_OUT), jnp.float32),
        scratch_shapes=[pltpu.VMEM((NUM_CLASSES, D_OUT), jnp.float32)],
    )(classes_embed, layers_feature, table, W1, W2, b2)
    return out
